# Initial kernel scaffold; baseline (speedup 1.0000x reference)
import jax
import jax.numpy as jnp
from jax.experimental import pallas as pl


def kernel(indices, labels, table):
    raise NotImplementedError("write your pallas kernel here")


# speedup vs baseline: 4.3140x; 4.3140x over previous
import jax
import jax.numpy as jnp
from jax import lax
from jax.experimental import pallas as pl
from jax.experimental.pallas import tpu as pltpu
from jax.experimental.pallas import tpu_sc as plsc

_L, _B, _D = 200, 4096, 64
_N = _L * _B
_NW = 32
_PER_W = _N // _NW
_CH = 800
_NCH = _PER_W // _CH


def _emb_body(table_hbm, idx_hbm, out_hbm, idx_v, rows_v, sem):
    wid = lax.axis_index("s") * 2 + lax.axis_index("c")
    base = wid * _PER_W
    pltpu.sync_copy(idx_hbm.at[pl.ds(base, _PER_W)], idx_v)

    def step(i, carry):
        off = i * _CH
        pltpu.async_copy(
            table_hbm.at[idx_v.at[pl.ds(off, _CH)]], rows_v, sem
        ).wait()
        pltpu.sync_copy(rows_v, out_hbm.at[pl.ds(base + off, _CH)])
        return carry

    lax.fori_loop(0, _NCH, step, 0)


def kernel(indices, labels, table):
    idx = indices.reshape(_N).astype(jnp.int32)
    mesh = plsc.VectorSubcoreMesh(core_axis_name="c", subcore_axis_name="s")
    out = pl.kernel(
        _emb_body,
        mesh=mesh,
        compiler_params=pltpu.CompilerParams(use_tc_tiling_on_sc=False),
        out_type=jax.ShapeDtypeStruct((_N, _D), jnp.float32),
        scratch_types=[
            pltpu.VMEM((_PER_W,), jnp.int32),
            pltpu.VMEM((_CH, _D), jnp.float32),
            pltpu.SemaphoreType.DMA,
        ],
    )(table, idx)
    return (out.reshape(_L, _B, _D), labels)


# speedup vs baseline: 4.3561x; 1.0098x over previous
import jax
import jax.numpy as jnp
from jax import lax
from jax.experimental import pallas as pl
from jax.experimental.pallas import tpu as pltpu
from jax.experimental.pallas import tpu_sc as plsc

_L, _B, _D = 200, 4096, 64
_N = _L * _B
_NW = 32
_PER_W = _N // _NW
_CH = 800
_NCH = _PER_W // _CH


def _emb_body(table_hbm, idx_hbm, out_hbm,
              idx_v, rows_a, rows_b, gsa, gsb, ssa, ssb):
    wid = lax.axis_index("s") * 2 + lax.axis_index("c")
    base = wid * _PER_W
    pltpu.sync_copy(idx_hbm.at[pl.ds(base, _PER_W)], idx_v)

    def gstart(c, buf, sem):
        pltpu.async_copy(table_hbm.at[idx_v.at[pl.ds(c * _CH, _CH)]], buf, sem)

    def gwait(c, buf, sem):
        pltpu.make_async_copy(
            table_hbm.at[idx_v.at[pl.ds(c * _CH, _CH)]], buf, sem
        ).wait()

    def sstart(c, buf, sem):
        pltpu.async_copy(buf, out_hbm.at[pl.ds(base + c * _CH, _CH)], sem)

    def swait(c, buf, sem):
        pltpu.make_async_copy(
            buf, out_hbm.at[pl.ds(base + c * _CH, _CH)], sem
        ).wait()

    gstart(0, rows_a, gsa)

    def step(j, carry):
        c0 = 2 * j
        c1 = c0 + 1
        gstart(c1, rows_b, gsb)
        gwait(c0, rows_a, gsa)
        sstart(c0, rows_a, ssa)
        gwait(c1, rows_b, gsb)
        sstart(c1, rows_b, ssb)
        swait(c0, rows_a, ssa)

        @pl.when(c0 + 2 < _NCH)
        def _():
            gstart(c0 + 2, rows_a, gsa)

        swait(c1, rows_b, ssb)
        return carry

    lax.fori_loop(0, _NCH // 2, step, 0)


def kernel(indices, labels, table):
    idx = indices.reshape(_N).astype(jnp.int32)
    mesh = plsc.VectorSubcoreMesh(core_axis_name="c", subcore_axis_name="s")
    out = pl.kernel(
        _emb_body,
        mesh=mesh,
        compiler_params=pltpu.CompilerParams(use_tc_tiling_on_sc=False),
        out_type=jax.ShapeDtypeStruct((_N, _D), jnp.float32),
        scratch_types=[
            pltpu.VMEM((_PER_W,), jnp.int32),
            pltpu.VMEM((_CH, _D), jnp.float32),
            pltpu.VMEM((_CH, _D), jnp.float32),
            pltpu.SemaphoreType.DMA,
            pltpu.SemaphoreType.DMA,
            pltpu.SemaphoreType.DMA,
            pltpu.SemaphoreType.DMA,
        ],
    )(table, idx)
    return (out.reshape(_L, _B, _D), labels)
